# baseline (speedup 1.0000x reference)
import functools

import jax
import jax.numpy as jnp
from jax import lax
from jax.experimental import pallas as pl
from jax.experimental.pallas import tpu as pltpu
from jax.experimental.pallas import tpu_sc as plsc

D_MODEL = 1024
VOCAB_ROWS = 50
B_TOTAL = 4 * 8192
_LANES = 16
_VPR = D_MODEL // _LANES

_INFO = plsc.get_sparse_core_info()
_NC = _INFO.num_cores
_NS = _INFO.num_subcores
_NW = _NC * _NS
_B_PER_W = B_TOTAL // _NW
_CHUNK = 32
_NCHUNK = _B_PER_W // _CHUNK


def _make_sc_lookup():
    mesh = plsc.VectorSubcoreMesh(core_axis_name="c", subcore_axis_name="s")

    @functools.partial(
        pl.kernel,
        mesh=mesh,
        out_type=jax.ShapeDtypeStruct((B_TOTAL, D_MODEL), jnp.float32),
        scratch_types=[
            pltpu.VMEM((_B_PER_W,), jnp.int32),
            pltpu.VMEM((VOCAB_ROWS, D_MODEL), jnp.float32),
            pltpu.VMEM((_CHUNK, D_MODEL), jnp.float32),
            pltpu.VMEM((_CHUNK, D_MODEL), jnp.float32),
            pltpu.SemaphoreType.DMA,
            pltpu.SemaphoreType.DMA,
        ],
    )
    def sc_lookup(table_hbm, idx_hbm, out_hbm, idx_v, table_v, rows_a, rows_b,
                  ssem_a, ssem_b):
        wid = lax.axis_index("s") * _NC + lax.axis_index("c")
        base = wid * _B_PER_W
        pltpu.sync_copy(table_hbm, table_v)
        pltpu.sync_copy(idx_hbm.at[pl.ds(base, _B_PER_W)], idx_v)

        def fill(c, buf):
            for g in range(_CHUNK // _LANES):
                r0 = g * _LANES
                idx16 = idx_v[pl.ds(c * _CHUNK + r0, _LANES)]
                rows = [idx16[l] for l in range(_LANES)]

                @plsc.parallel_loop(0, _VPR, unroll=4)
                def _(j):
                    col = pl.ds(j * _LANES, _LANES)
                    for l in range(_LANES):
                        buf[r0 + l, col] = table_v[rows[l], col]

        def scatter_start(c, buf, sem):
            pltpu.async_copy(
                buf, out_hbm.at[pl.ds(base + c * _CHUNK, _CHUNK)], sem
            )

        def scatter_wait(buf, sem):
            pltpu.make_async_copy(
                buf, out_hbm.at[pl.ds(base, _CHUNK)], sem
            ).wait()

        @pl.loop(0, _NCHUNK, step=2)
        def _(i):
            @pl.when(i > 0)
            def _():
                scatter_wait(rows_a, ssem_a)

            fill(i, rows_a)

            @pl.when(i > 0)
            def _():
                scatter_wait(rows_b, ssem_b)

            scatter_start(i, rows_a, ssem_a)
            fill(i + 1, rows_b)
            scatter_start(i + 1, rows_b, ssem_b)

        scatter_wait(rows_a, ssem_a)
        scatter_wait(rows_b, ssem_b)

    return sc_lookup


_sc_lookup = _make_sc_lookup()


@jax.jit
def kernel(x, table):
    flat_idx = x.reshape(B_TOTAL).astype(jnp.int32)
    out = _sc_lookup(table, flat_idx)
    return out.reshape(x.shape[0], x.shape[1], D_MODEL)


# speedup vs baseline: 1.0240x; 1.0240x over previous
import functools

import jax
import jax.numpy as jnp
from jax import lax
from jax.experimental import pallas as pl
from jax.experimental.pallas import tpu as pltpu
from jax.experimental.pallas import tpu_sc as plsc

D_MODEL = 1024
VOCAB_ROWS = 50
B_TOTAL = 4 * 8192
_LANES = 16
_VPR = D_MODEL // _LANES

_INFO = plsc.get_sparse_core_info()
_NC = _INFO.num_cores
_NS = _INFO.num_subcores
_NW = _NC * _NS
_B_PER_W = B_TOTAL // _NW
_CHUNK = 32
_NCHUNK = _B_PER_W // _CHUNK


def _make_sc_lookup():
    mesh = plsc.VectorSubcoreMesh(core_axis_name="c", subcore_axis_name="s")

    @functools.partial(
        pl.kernel,
        mesh=mesh,
        out_type=jax.ShapeDtypeStruct((B_TOTAL, D_MODEL), jnp.float32),
        scratch_types=[
            pltpu.VMEM((_B_PER_W,), jnp.int32),
            pltpu.VMEM((VOCAB_ROWS, D_MODEL), jnp.float32),
            pltpu.VMEM((_CHUNK, D_MODEL), jnp.float32),
            pltpu.VMEM((_CHUNK, D_MODEL), jnp.float32),
            pltpu.SemaphoreType.DMA,
            pltpu.SemaphoreType.DMA,
        ],
    )
    def sc_lookup(table_hbm, idx_hbm, out_hbm, idx_v, table_v, rows_a, rows_b,
                  ssem_a, ssem_b):
        wid = lax.axis_index("s") * _NC + lax.axis_index("c")
        base = wid * _B_PER_W
        pltpu.sync_copy(table_hbm, table_v)
        pltpu.sync_copy(idx_hbm.at[pl.ds(base, _B_PER_W)], idx_v)

        def fill(c, buf):
            for g in range(_CHUNK // _LANES):
                r0 = g * _LANES
                idx16 = idx_v[pl.ds(c * _CHUNK + r0, _LANES)]
                rows = [idx16[l] for l in range(_LANES)]

                @plsc.parallel_loop(0, _VPR, unroll=1)
                def _(j):
                    col = pl.ds(j * _LANES, _LANES)
                    for l in range(_LANES):
                        buf[r0 + l, col] = table_v[rows[l], col]

        def scatter_start(c, buf, sem):
            pltpu.async_copy(
                buf, out_hbm.at[pl.ds(base + c * _CHUNK, _CHUNK)], sem
            )

        def scatter_wait(buf, sem):
            pltpu.make_async_copy(
                buf, out_hbm.at[pl.ds(base, _CHUNK)], sem
            ).wait()

        @pl.loop(0, _NCHUNK, step=2)
        def _(i):
            @pl.when(i > 0)
            def _():
                scatter_wait(rows_a, ssem_a)

            fill(i, rows_a)

            @pl.when(i > 0)
            def _():
                scatter_wait(rows_b, ssem_b)

            scatter_start(i, rows_a, ssem_a)
            fill(i + 1, rows_b)
            scatter_start(i + 1, rows_b, ssem_b)

        scatter_wait(rows_a, ssem_a)
        scatter_wait(rows_b, ssem_b)

    return sc_lookup


_sc_lookup = _make_sc_lookup()


@jax.jit
def kernel(x, table):
    flat_idx = x.reshape(B_TOTAL).astype(jnp.int32)
    out = _sc_lookup(table, flat_idx)
    return out.reshape(x.shape[0], x.shape[1], D_MODEL)


# speedup vs baseline: 1.3617x; 1.3298x over previous
import functools

import jax
import jax.numpy as jnp
from jax import lax
from jax.experimental import pallas as pl
from jax.experimental.pallas import tpu as pltpu
from jax.experimental.pallas import tpu_sc as plsc

D_MODEL = 1024
VOCAB_ROWS = 50
B_TOTAL = 4 * 8192
_LANES = 16

_INFO = plsc.get_sparse_core_info()
_NC = _INFO.num_cores
_NS = _INFO.num_subcores
_NW = _NC * _NS
_B_PER_W = B_TOTAL // _NW
_NGROUP = _B_PER_W // _LANES


def _make_sc_lookup():
    mesh = plsc.VectorSubcoreMesh(core_axis_name="c", subcore_axis_name="s")

    @functools.partial(
        pl.kernel,
        mesh=mesh,
        out_type=jax.ShapeDtypeStruct((B_TOTAL, D_MODEL), jnp.float32),
        scratch_types=[
            pltpu.VMEM((_B_PER_W,), jnp.int32),
            pltpu.VMEM((VOCAB_ROWS, D_MODEL), jnp.float32),
            pltpu.SemaphoreType.DMA,
        ],
    )
    def sc_lookup(table_hbm, idx_hbm, out_hbm, idx_v, table_v, sem):
        wid = lax.axis_index("s") * _NC + lax.axis_index("c")
        base = wid * _B_PER_W
        pltpu.sync_copy(table_hbm, table_v)
        pltpu.sync_copy(idx_hbm.at[pl.ds(base, _B_PER_W)], idx_v)

        @pl.loop(0, _NGROUP)
        def _(g):
            r0 = g * _LANES
            idx16 = idx_v[pl.ds(r0, _LANES)]
            for l in range(_LANES):
                pltpu.async_copy(
                    table_v.at[idx16[l]], out_hbm.at[base + r0 + l], sem
                )

        @pl.loop(0, _NGROUP)
        def _(g):
            pltpu.make_async_copy(
                table_v.at[pl.ds(0, _LANES)],
                out_hbm.at[pl.ds(base, _LANES)],
                sem,
            ).wait()

    return sc_lookup


_sc_lookup = _make_sc_lookup()


@jax.jit
def kernel(x, table):
    flat_idx = x.reshape(B_TOTAL).astype(jnp.int32)
    out = _sc_lookup(table, flat_idx)
    return out.reshape(x.shape[0], x.shape[1], D_MODEL)
